# baseline (speedup 1.0000x reference)
import functools

import jax
import jax.numpy as jnp
from jax import lax
from jax.experimental import pallas as pl
from jax.experimental.pallas import tpu as pltpu
from jax.experimental.pallas import tpu_sc as plsc

HIDDEN = 768
EPS = 1e-5
L = 16
NSLICE = HIDDEN // L
NC, NS = 2, 16
NW = NC * NS
CHUNK = 64


def _rsqrt16(x):
    i = lax.bitcast_convert_type(x, jnp.int32)
    i = jnp.int32(0x5F3759DF) - lax.shift_right_logical(i, jnp.int32(1))
    y = lax.bitcast_convert_type(i, jnp.float32)
    for _ in range(3):
        y = y * (jnp.float32(1.5) - jnp.float32(0.5) * x * y * y)
    return y


def _allsum16(x):
    lanes = lax.iota(jnp.int32, L)
    dnums = lax.GatherDimensionNumbers(
        offset_dims=(), collapsed_slice_dims=(0,), start_index_map=(0,))
    for k in (1, 2, 4, 8):
        idx = (lanes ^ k).reshape(L, 1)
        x = x + lax.gather(x, idx, dnums, slice_sizes=(1,),
                           mode=lax.GatherScatterMode.PROMISE_IN_BOUNDS)
    return x


def _normalize_chunk(rows_v, buf, w_v):
    inv_h = jnp.float32(1.0 / HIDDEN)

    def row_body(r, _):
        s = jnp.zeros((L,), jnp.float32)
        q = jnp.zeros((L,), jnp.float32)
        for j in range(NSLICE):
            v = rows_v[buf, r, pl.ds(j * L, L)]
            s = s + v
            q = q + v * v
        mean_v = _allsum16(s) * inv_h
        msq_v = _allsum16(q) * inv_h
        var_v = msq_v - mean_v * mean_v
        rstd_v = _rsqrt16(var_v + jnp.float32(EPS))
        for j in range(NSLICE):
            v = rows_v[buf, r, pl.ds(j * L, L)]
            w = w_v[pl.ds(j * L, L)]
            rows_v[buf, r, pl.ds(j * L, L)] = (v - mean_v) * (rstd_v * w)
        return 0

    lax.fori_loop(0, CHUNK, row_body, 0)


def _build_sc_kernel(B):
    b_per_w = B // NW
    n_chunks = b_per_w // CHUNK
    mesh = plsc.VectorSubcoreMesh(core_axis_name="c", subcore_axis_name="s")

    @functools.partial(
        pl.kernel,
        mesh=mesh,
        out_type=jax.ShapeDtypeStruct((B, HIDDEN), jnp.float32),
        scratch_types=[
            pltpu.VMEM((n_chunks, CHUNK), jnp.int32),
            pltpu.VMEM((2, CHUNK, HIDDEN), jnp.float32),
            pltpu.VMEM((HIDDEN,), jnp.float32),
            pltpu.SemaphoreType.DMA,
            pltpu.SemaphoreType.DMA,
            pltpu.SemaphoreType.DMA,
        ],
    )
    def k(ids_hbm, table_hbm, w_hbm, out_hbm, idx_v, rows_v, w_v, gsem0, gsem1, osem):
        wid = lax.axis_index("s") * NC + lax.axis_index("c")
        base = wid * b_per_w
        pltpu.sync_copy(w_hbm, w_v)
        pltpu.sync_copy(ids_hbm.at[wid], idx_v)

        gsems = (gsem0, gsem1)

        def gather(ci, buf):
            return pltpu.async_copy(
                table_hbm.at[idx_v.at[ci]], rows_v.at[buf], gsems[buf])

        gather(0, 0).wait()
        for ci in range(n_chunks):
            buf = ci % 2
            if ci + 1 < n_chunks:
                nxt = gather(ci + 1, 1 - buf)
            if ci > 1:
                pass
            _normalize_chunk(rows_v, buf, w_v)
            st = pltpu.async_copy(
                rows_v.at[buf], out_hbm.at[pl.ds(base + ci * CHUNK, CHUNK)], osem)
            st.wait()
            if ci + 1 < n_chunks:
                nxt.wait()

    return k


@jax.jit
def kernel(input_ids, tok_embeddings, norm_weight):
    B_, S_ = input_ids.shape
    B = B_ * S_
    ids3 = input_ids.astype(jnp.int32).reshape(NW, (B // NW) // CHUNK, CHUNK)
    k = _build_sc_kernel(B)
    out = k(ids3, tok_embeddings, norm_weight)
    return out.reshape(B_, S_, HIDDEN)
